# baseline (speedup 1.0000x reference)
import functools

import jax
import jax.numpy as jnp
from jax import lax
from jax.experimental import pallas as pl
from jax.experimental.pallas import tpu as pltpu
from jax.experimental.pallas import tpu_sc as plsc

D_MODEL_K = 2048
N_EXPERTS = 16
K_TOP = 2
N_TOK = 16384
BLOCK = 2048

NC, NS = 2, 16
NW = NC * NS
TOK_PER_TILE = N_TOK // NW
GROUPS = TOK_PER_TILE // 16

_mesh = plsc.VectorSubcoreMesh(
    core_axis_name="c", subcore_axis_name="s",
    num_cores=NC, num_subcores=NS)


def _mm_body(x_ref, w_ref, b_ref, lg_ref):
    lg_ref[...] = jax.lax.dot_general(
        x_ref[...], w_ref[...],
        dimension_numbers=(((1,), (1,)), ((), ())),
        preferred_element_type=jnp.float32,
    ) + b_ref[...]


def _tc_logits(x, W, b2):
    n_tok = x.shape[0]
    return pl.pallas_call(
        _mm_body,
        grid=(n_tok // BLOCK,),
        in_specs=[
            pl.BlockSpec((BLOCK, D_MODEL_K), lambda i: (i, 0)),
            pl.BlockSpec((N_EXPERTS, D_MODEL_K), lambda i: (0, 0)),
            pl.BlockSpec((1, N_EXPERTS), lambda i: (0, 0)),
        ],
        out_specs=pl.BlockSpec((BLOCK, N_EXPERTS), lambda i: (i, 0)),
        out_shape=jax.ShapeDtypeStruct((n_tok, N_EXPERTS), jnp.float32),
        compiler_params=pltpu.CompilerParams(
            dimension_semantics=("arbitrary",),
        ),
    )(x, W, b2)


@functools.partial(
    pl.kernel,
    out_type=(
        jax.ShapeDtypeStruct((N_TOK * K_TOP,), jnp.float32),
        jax.ShapeDtypeStruct((N_TOK * K_TOP,), jnp.int32),
        jax.ShapeDtypeStruct((N_TOK * N_EXPERTS,), jnp.float32),
    ),
    mesh=_mesh,
    scratch_types=[
        pltpu.VMEM((TOK_PER_TILE * N_EXPERTS,), jnp.float32),
        pltpu.VMEM((TOK_PER_TILE * K_TOP + 16,), jnp.float32),
        pltpu.VMEM((TOK_PER_TILE * K_TOP + 16,), jnp.int32),
        pltpu.VMEM((TOK_PER_TILE * N_EXPERTS,), jnp.float32),
    ],
    compiler_params=pltpu.CompilerParams(needs_layout_passes=False),
)
def _sc_route(lg_hbm, tw_hbm, ti_hbm, wt_hbm, lbuf, twb, tib, wtb):
    wid = lax.axis_index("s") * NC + lax.axis_index("c")
    base = wid * TOK_PER_TILE
    pltpu.sync_copy(
        lg_hbm.at[pl.ds(base * N_EXPERTS, TOK_PER_TILE * N_EXPERTS)], lbuf)

    lane = lax.iota(jnp.int32, 16)
    mask2 = lane < K_TOP
    UNROLL = 4

    def one_token(t):
        v = lbuf[pl.ds(t * N_EXPERTS, N_EXPERTS)]
        m = jnp.max(v)
        e_x = jnp.exp(v - m)
        s = jnp.sum(e_x)
        w = e_x / s
        wtb[pl.ds(t * N_EXPERTS, N_EXPERTS)] = w
        sw, si = plsc.sort_key_val(w, lane, descending=True)
        plsc.store_compressed(twb.at[pl.ds(t * K_TOP, 16)], sw, mask=mask2)
        plsc.store_compressed(tib.at[pl.ds(t * K_TOP, 16)], si, mask=mask2)

    def group_body(g, carry):
        t0 = g * UNROLL
        for u in range(UNROLL):
            one_token(t0 + u)
        return carry

    lax.fori_loop(0, TOK_PER_TILE // UNROLL, group_body, jnp.int32(0))

    pltpu.sync_copy(twb.at[pl.ds(0, TOK_PER_TILE * K_TOP)],
                    tw_hbm.at[pl.ds(base * K_TOP, TOK_PER_TILE * K_TOP)])
    pltpu.sync_copy(tib.at[pl.ds(0, TOK_PER_TILE * K_TOP)],
                    ti_hbm.at[pl.ds(base * K_TOP, TOK_PER_TILE * K_TOP)])
    pltpu.sync_copy(
        wtb, wt_hbm.at[pl.ds(base * N_EXPERTS, TOK_PER_TILE * N_EXPERTS)])


@functools.partial(jax.jit, static_argnames=())
def kernel(x, W, b):
    n_tok = x.shape[0]
    b2 = b.reshape(1, N_EXPERTS)
    logits = _tc_logits(x, W, b2)
    tw, ti, wts = _sc_route(logits.reshape(-1))
    return (
        tw.reshape(n_tok, K_TOP),
        ti.reshape(n_tok, K_TOP),
        wts.reshape(n_tok, N_EXPERTS),
    )


# speedup vs baseline: 1.1383x; 1.1383x over previous
import functools

import jax
import jax.numpy as jnp
from jax import lax
from jax.experimental import pallas as pl
from jax.experimental.pallas import tpu as pltpu
from jax.experimental.pallas import tpu_sc as plsc

D_MODEL_K = 2048
N_EXPERTS = 16
K_TOP = 2
N_TOK = 16384
BLOCK = 2048

NC, NS = 2, 16
NW = NC * NS
TOK_PER_TILE = N_TOK // NW
GROUPS = TOK_PER_TILE // 16

_mesh = plsc.VectorSubcoreMesh(
    core_axis_name="c", subcore_axis_name="s",
    num_cores=NC, num_subcores=NS)


def _mm_body(x_ref, w_ref, b_ref, lg_ref):
    lg_ref[...] = jax.lax.dot_general(
        x_ref[...], w_ref[...],
        dimension_numbers=(((1,), (1,)), ((), ())),
        preferred_element_type=jnp.float32,
    ) + b_ref[...]


def _tc_logits(x, W, b2):
    n_tok = x.shape[0]
    return pl.pallas_call(
        _mm_body,
        grid=(n_tok // BLOCK,),
        in_specs=[
            pl.BlockSpec((BLOCK, D_MODEL_K), lambda i: (i, 0)),
            pl.BlockSpec((N_EXPERTS, D_MODEL_K), lambda i: (0, 0)),
            pl.BlockSpec((1, N_EXPERTS), lambda i: (0, 0)),
        ],
        out_specs=pl.BlockSpec((BLOCK, N_EXPERTS), lambda i: (i, 0)),
        out_shape=jax.ShapeDtypeStruct((n_tok, N_EXPERTS), jnp.float32),
        compiler_params=pltpu.CompilerParams(
            dimension_semantics=("arbitrary",),
        ),
    )(x, W, b2)


@functools.partial(
    pl.kernel,
    out_type=(
        jax.ShapeDtypeStruct((N_TOK * K_TOP,), jnp.float32),
        jax.ShapeDtypeStruct((N_TOK * K_TOP,), jnp.int32),
        jax.ShapeDtypeStruct((N_TOK * N_EXPERTS,), jnp.float32),
    ),
    mesh=_mesh,
    scratch_types=[
        pltpu.VMEM((TOK_PER_TILE * N_EXPERTS,), jnp.float32),
        pltpu.VMEM((TOK_PER_TILE * K_TOP + 16,), jnp.float32),
        pltpu.VMEM((TOK_PER_TILE * K_TOP + 16,), jnp.int32),
        pltpu.VMEM((TOK_PER_TILE * N_EXPERTS,), jnp.float32),
    ],
    compiler_params=pltpu.CompilerParams(needs_layout_passes=False),
)
def _sc_route(lg_hbm, tw_hbm, ti_hbm, wt_hbm, lbuf, twb, tib, wtb):
    wid = lax.axis_index("s") * NC + lax.axis_index("c")
    base = wid * TOK_PER_TILE
    pltpu.sync_copy(
        lg_hbm.at[pl.ds(base * N_EXPERTS, TOK_PER_TILE * N_EXPERTS)], lbuf)

    lane = lax.iota(jnp.int32, 16)
    lane16 = lane * N_EXPERTS
    lane2 = lane * K_TOP
    neg_inf = jnp.full((16,), -jnp.inf, jnp.float32)
    zero_f = jnp.zeros((16,), jnp.float32)

    def group_body(g, carry):
        gbase = g * (16 * N_EXPERTS) + lane16
        L = [plsc.load_gather(lbuf, [gbase + e]) for e in range(N_EXPERTS)]

        m = L[0]
        for e in range(1, N_EXPERTS):
            m = jnp.maximum(m, L[e])
        exps = [jnp.exp(L[e] - m) for e in range(N_EXPERTS)]
        s = exps[0]
        for e in range(1, N_EXPERTS):
            s = s + exps[e]
        r = 1.0 / s

        m1, i1 = L[0], zero_f
        m2, i2 = neg_inf, zero_f
        for e in range(1, N_EXPERTS):
            e_f = jnp.full((16,), float(e), jnp.float32)
            gt1 = L[e] > m1
            gt2 = L[e] > m2
            m2 = jnp.where(gt1, m1, jnp.where(gt2, L[e], m2))
            i2 = jnp.where(gt1, i1, jnp.where(gt2, e_f, i2))
            m1 = jnp.where(gt1, L[e], m1)
            i1 = jnp.where(gt1, e_f, i1)

        w1 = jnp.exp(m1 - m) * r
        w2 = jnp.exp(m2 - m) * r

        for e in range(N_EXPERTS):
            plsc.store_scatter(wtb, [gbase + e], exps[e] * r)
        tbase = g * (16 * K_TOP) + lane2
        plsc.store_scatter(twb, [tbase], w1)
        plsc.store_scatter(twb, [tbase + 1], w2)
        plsc.store_scatter(tib, [tbase], i1.astype(jnp.int32))
        plsc.store_scatter(tib, [tbase + 1], i2.astype(jnp.int32))
        return carry

    lax.fori_loop(0, GROUPS, group_body, jnp.int32(0))

    pltpu.sync_copy(twb.at[pl.ds(0, TOK_PER_TILE * K_TOP)],
                    tw_hbm.at[pl.ds(base * K_TOP, TOK_PER_TILE * K_TOP)])
    pltpu.sync_copy(tib.at[pl.ds(0, TOK_PER_TILE * K_TOP)],
                    ti_hbm.at[pl.ds(base * K_TOP, TOK_PER_TILE * K_TOP)])
    pltpu.sync_copy(
        wtb, wt_hbm.at[pl.ds(base * N_EXPERTS, TOK_PER_TILE * N_EXPERTS)])


@functools.partial(jax.jit, static_argnames=())
def kernel(x, W, b):
    n_tok = x.shape[0]
    b2 = b.reshape(1, N_EXPERTS)
    logits = _tc_logits(x, W, b2)
    tw, ti, wts = _sc_route(logits.reshape(-1))
    return (
        tw.reshape(n_tok, K_TOP),
        ti.reshape(n_tok, K_TOP),
        wts.reshape(n_tok, N_EXPERTS),
    )


# speedup vs baseline: 1.2245x; 1.0757x over previous
import functools

import jax
import jax.numpy as jnp
from jax import lax
from jax.experimental import pallas as pl
from jax.experimental.pallas import tpu as pltpu
from jax.experimental.pallas import tpu_sc as plsc

D_MODEL_K = 2048
N_EXPERTS = 16
K_TOP = 2
N_TOK = 16384
BLOCK = 2048

NC, NS = 2, 16
NW = NC * NS
TOK_PER_TILE = N_TOK // NW
GROUPS = TOK_PER_TILE // 16

_mesh = plsc.VectorSubcoreMesh(
    core_axis_name="c", subcore_axis_name="s",
    num_cores=NC, num_subcores=NS)


def _mm_body(x_ref, w_ref, b_ref, lg_ref):
    lg_ref[...] = jax.lax.dot_general(
        x_ref[...], w_ref[...],
        dimension_numbers=(((1,), (1,)), ((), ())),
        preferred_element_type=jnp.float32,
    ) + b_ref[...]


def _tc_logits(x, W, b2):
    n_tok = x.shape[0]
    return pl.pallas_call(
        _mm_body,
        grid=(n_tok // BLOCK,),
        in_specs=[
            pl.BlockSpec((BLOCK, D_MODEL_K), lambda i: (i, 0)),
            pl.BlockSpec((N_EXPERTS, D_MODEL_K), lambda i: (0, 0)),
            pl.BlockSpec((1, N_EXPERTS), lambda i: (0, 0)),
        ],
        out_specs=pl.BlockSpec((BLOCK, N_EXPERTS), lambda i: (i, 0)),
        out_shape=jax.ShapeDtypeStruct((n_tok, N_EXPERTS), jnp.float32),
        compiler_params=pltpu.CompilerParams(
            dimension_semantics=("arbitrary",),
        ),
    )(x, W, b2)


@functools.partial(
    pl.kernel,
    out_type=(
        jax.ShapeDtypeStruct((NW, TOK_PER_TILE, K_TOP), jnp.float32),
        jax.ShapeDtypeStruct((NW, TOK_PER_TILE, K_TOP), jnp.int32),
        jax.ShapeDtypeStruct((NW, TOK_PER_TILE, N_EXPERTS), jnp.float32),
    ),
    mesh=_mesh,
    scratch_types=[
        pltpu.VMEM((TOK_PER_TILE, N_EXPERTS), jnp.float32),
        pltpu.VMEM((TOK_PER_TILE, K_TOP), jnp.float32),
        pltpu.VMEM((TOK_PER_TILE, K_TOP), jnp.int32),
        pltpu.VMEM((TOK_PER_TILE, N_EXPERTS), jnp.float32),
    ],
    compiler_params=pltpu.CompilerParams(
        needs_layout_passes=False, use_tc_tiling_on_sc=False),
)
def _sc_route(lg_hbm, tw_hbm, ti_hbm, wt_hbm, lbuf, twb, tib, wtb):
    wid = lax.axis_index("s") * NC + lax.axis_index("c")
    base = wid * TOK_PER_TILE
    pltpu.sync_copy(lg_hbm.at[wid], lbuf)

    lane = lax.iota(jnp.int32, 16)
    neg_inf = jnp.full((16,), -jnp.inf, jnp.float32)
    zero_f = jnp.zeros((16,), jnp.float32)
    zero_i = jnp.zeros((16,), jnp.int32)
    one_i = jnp.full((16,), 1, jnp.int32)

    def group_body(g, carry):
        row = g * 16 + lane
        L = [plsc.load_gather(lbuf, [row, jnp.full((16,), e, jnp.int32)])
             for e in range(N_EXPERTS)]

        m = L[0]
        for e in range(1, N_EXPERTS):
            m = jnp.maximum(m, L[e])
        exps = [jnp.exp(L[e] - m) for e in range(N_EXPERTS)]
        s = exps[0]
        for e in range(1, N_EXPERTS):
            s = s + exps[e]
        r = 1.0 / s

        m1, i1 = L[0], zero_f
        m2, i2 = neg_inf, zero_f
        for e in range(1, N_EXPERTS):
            e_f = jnp.full((16,), float(e), jnp.float32)
            gt1 = L[e] > m1
            gt2 = L[e] > m2
            m2 = jnp.where(gt1, m1, jnp.where(gt2, L[e], m2))
            i2 = jnp.where(gt1, i1, jnp.where(gt2, e_f, i2))
            m1 = jnp.where(gt1, L[e], m1)
            i1 = jnp.where(gt1, e_f, i1)

        w1 = jnp.exp(m1 - m) * r
        w2 = jnp.exp(m2 - m) * r

        for e in range(N_EXPERTS):
            plsc.store_scatter(
                wtb, [row, jnp.full((16,), e, jnp.int32)], exps[e] * r)
        plsc.store_scatter(twb, [row, zero_i], w1)
        plsc.store_scatter(twb, [row, one_i], w2)
        plsc.store_scatter(tib, [row, zero_i], i1.astype(jnp.int32))
        plsc.store_scatter(tib, [row, one_i], i2.astype(jnp.int32))
        return carry

    lax.fori_loop(0, GROUPS, group_body, jnp.int32(0))

    pltpu.sync_copy(twb, tw_hbm.at[wid])
    pltpu.sync_copy(tib, ti_hbm.at[wid])
    pltpu.sync_copy(wtb, wt_hbm.at[wid])


@functools.partial(jax.jit, static_argnames=())
def kernel(x, W, b):
    n_tok = x.shape[0]
    b2 = b.reshape(1, N_EXPERTS)
    logits = _tc_logits(x, W, b2)
    tw, ti, wts = _sc_route(
        logits.reshape(NW, TOK_PER_TILE, N_EXPERTS))
    return (
        tw.reshape(n_tok, K_TOP),
        ti.reshape(n_tok, K_TOP),
        wts.reshape(n_tok, N_EXPERTS),
    )


# speedup vs baseline: 1.3281x; 1.0846x over previous
import functools

import jax
import jax.numpy as jnp
from jax import lax
from jax.experimental import pallas as pl
from jax.experimental.pallas import tpu as pltpu
from jax.experimental.pallas import tpu_sc as plsc

D_MODEL_K = 2048
N_EXPERTS = 16
K_TOP = 2
N_TOK = 16384
BLOCK = 2048

NC, NS = 2, 16
NW = NC * NS
TOK_PER_TILE = N_TOK // NW
CHUNK_T = 256
N_CHUNKS = TOK_PER_TILE // CHUNK_T
CGROUPS = CHUNK_T // 16

_mesh = plsc.VectorSubcoreMesh(
    core_axis_name="c", subcore_axis_name="s",
    num_cores=NC, num_subcores=NS)


def _mm_body(x_ref, w_ref, b_ref, lg_ref):
    lg_ref[...] = jax.lax.dot_general(
        x_ref[...], w_ref[...],
        dimension_numbers=(((1,), (1,)), ((), ())),
        preferred_element_type=jnp.float32,
    ) + b_ref[...]


def _tc_logits(x, W, b2):
    n_tok = x.shape[0]
    return pl.pallas_call(
        _mm_body,
        grid=(n_tok // BLOCK,),
        in_specs=[
            pl.BlockSpec((BLOCK, D_MODEL_K), lambda i: (i, 0)),
            pl.BlockSpec((N_EXPERTS, D_MODEL_K), lambda i: (0, 0)),
            pl.BlockSpec((1, N_EXPERTS), lambda i: (0, 0)),
        ],
        out_specs=pl.BlockSpec((BLOCK, N_EXPERTS), lambda i: (i, 0)),
        out_shape=jax.ShapeDtypeStruct((n_tok, N_EXPERTS), jnp.float32),
        compiler_params=pltpu.CompilerParams(
            dimension_semantics=("arbitrary",),
        ),
    )(x, W, b2)


@functools.partial(
    pl.kernel,
    out_type=(
        jax.ShapeDtypeStruct((N_TOK, K_TOP), jnp.float32),
        jax.ShapeDtypeStruct((N_TOK, K_TOP), jnp.int32),
        jax.ShapeDtypeStruct((N_TOK, N_EXPERTS), jnp.float32),
    ),
    mesh=_mesh,
    scratch_types=[
        pltpu.VMEM((CHUNK_T, N_EXPERTS), jnp.float32),
        pltpu.VMEM((CHUNK_T, K_TOP), jnp.float32),
        pltpu.VMEM((CHUNK_T, K_TOP), jnp.int32),
    ],
    compiler_params=pltpu.CompilerParams(
        needs_layout_passes=False, use_tc_tiling_on_sc=True),
)
def _sc_route(lg_hbm, tw_hbm, ti_hbm, wt_hbm, lbuf, twb, tib):
    wid = lax.axis_index("s") * NC + lax.axis_index("c")

    lane = lax.iota(jnp.int32, 16)
    neg_inf = jnp.full((16,), -jnp.inf, jnp.float32)
    zero_f = jnp.zeros((16,), jnp.float32)
    zero_i = jnp.zeros((16,), jnp.int32)
    one_i = jnp.full((16,), 1, jnp.int32)

    def group_body(g, carry):
        row = g * 16 + lane
        L = [plsc.load_gather(lbuf, [row, jnp.full((16,), e, jnp.int32)])
             for e in range(N_EXPERTS)]

        m = L[0]
        for e in range(1, N_EXPERTS):
            m = jnp.maximum(m, L[e])
        exps = [jnp.exp(L[e] - m) for e in range(N_EXPERTS)]
        s = exps[0]
        for e in range(1, N_EXPERTS):
            s = s + exps[e]
        r = 1.0 / s

        m1, i1 = L[0], zero_f
        m2, i2 = neg_inf, zero_f
        for e in range(1, N_EXPERTS):
            e_f = jnp.full((16,), float(e), jnp.float32)
            gt1 = L[e] > m1
            gt2 = L[e] > m2
            m2 = jnp.where(gt1, m1, jnp.where(gt2, L[e], m2))
            i2 = jnp.where(gt1, i1, jnp.where(gt2, e_f, i2))
            m1 = jnp.where(gt1, L[e], m1)
            i1 = jnp.where(gt1, e_f, i1)

        w1 = jnp.exp(m1 - m) * r
        w2 = jnp.exp(m2 - m) * r

        for e in range(N_EXPERTS):
            plsc.store_scatter(
                lbuf, [row, jnp.full((16,), e, jnp.int32)], exps[e] * r)
        plsc.store_scatter(twb, [row, zero_i], w1)
        plsc.store_scatter(twb, [row, one_i], w2)
        plsc.store_scatter(tib, [row, zero_i], i1.astype(jnp.int32))
        plsc.store_scatter(tib, [row, one_i], i2.astype(jnp.int32))
        return carry

    for c in range(N_CHUNKS):
        base = wid * TOK_PER_TILE + c * CHUNK_T
        pltpu.sync_copy(lg_hbm.at[pl.ds(base, CHUNK_T), :], lbuf)
        lax.fori_loop(0, CGROUPS, group_body, jnp.int32(0))
        pltpu.sync_copy(twb, tw_hbm.at[pl.ds(base, CHUNK_T), :])
        pltpu.sync_copy(tib, ti_hbm.at[pl.ds(base, CHUNK_T), :])
        pltpu.sync_copy(lbuf, wt_hbm.at[pl.ds(base, CHUNK_T), :])


@functools.partial(jax.jit, static_argnames=())
def kernel(x, W, b):
    b2 = b.reshape(1, N_EXPERTS)
    logits = _tc_logits(x, W, b2)
    tw, ti, wts = _sc_route(logits)
    return (tw, ti, wts)


# speedup vs baseline: 1.9055x; 1.4348x over previous
import functools

import jax
import jax.numpy as jnp
from jax.experimental import pallas as pl
from jax.experimental.pallas import tpu as pltpu

D_MODEL_K = 2048
N_EXPERTS = 16
K_TOP = 2
BLOCK = 2048


def _gate_body(x_ref, w_ref, b_ref, tw_ref, ti_ref, wout_ref):
    logits = jax.lax.dot_general(
        x_ref[...], w_ref[...],
        dimension_numbers=(((1,), (1,)), ((), ())),
        preferred_element_type=jnp.float32,
    ) + b_ref[...]

    m1 = jnp.max(logits, axis=-1, keepdims=True)
    e = jnp.exp(logits - m1)
    s = jnp.sum(e, axis=-1, keepdims=True)
    wts = e / s

    iota_f = jax.lax.broadcasted_iota(
        jnp.int32, logits.shape, 1).astype(jnp.float32)
    big_f = jnp.float32(N_EXPERTS)
    neg_inf = jnp.float32(-jnp.inf)

    i1_f = jnp.min(jnp.where(logits == m1, iota_f, big_f), axis=-1,
                   keepdims=True)
    logits2 = jnp.where(iota_f == i1_f, neg_inf, logits)
    m2 = jnp.max(logits2, axis=-1, keepdims=True)
    i2_f = jnp.min(jnp.where(logits2 == m2, iota_f, big_f), axis=-1,
                   keepdims=True)

    w1 = jnp.max(wts, axis=-1, keepdims=True)
    w2 = jnp.max(jnp.where(iota_f == i1_f, jnp.float32(0.0), wts),
                 axis=-1, keepdims=True)

    tw_ref[...] = jnp.concatenate([w1, w2], axis=-1)
    ti_ref[...] = jnp.concatenate([i1_f, i2_f], axis=-1).astype(jnp.int32)
    wout_ref[...] = wts


@functools.partial(jax.jit, static_argnames=())
def kernel(x, W, b):
    n_tok = x.shape[0]
    grid = (n_tok // BLOCK,)
    b2 = b.reshape(1, N_EXPERTS)
    out_shapes = (
        jax.ShapeDtypeStruct((n_tok, K_TOP), jnp.float32),
        jax.ShapeDtypeStruct((n_tok, K_TOP), jnp.int32),
        jax.ShapeDtypeStruct((n_tok, N_EXPERTS), jnp.float32),
    )
    tw, ti, wts = pl.pallas_call(
        _gate_body,
        grid=grid,
        in_specs=[
            pl.BlockSpec((BLOCK, D_MODEL_K), lambda i: (i, 0)),
            pl.BlockSpec((N_EXPERTS, D_MODEL_K), lambda i: (0, 0)),
            pl.BlockSpec((1, N_EXPERTS), lambda i: (0, 0)),
        ],
        out_specs=[
            pl.BlockSpec((BLOCK, K_TOP), lambda i: (i, 0)),
            pl.BlockSpec((BLOCK, K_TOP), lambda i: (i, 0)),
            pl.BlockSpec((BLOCK, N_EXPERTS), lambda i: (i, 0)),
        ],
        out_shape=out_shapes,
        compiler_params=pltpu.CompilerParams(
            dimension_semantics=("arbitrary",),
        ),
    )(x, W, b2)
    return (tw, ti, wts)
